# baseline (speedup 1.0000x reference)
import functools

import jax
import jax.numpy as jnp
from jax import lax
from jax.experimental import pallas as pl
from jax.experimental.pallas import tpu as pltpu
from jax.experimental.pallas import tpu_sc as plsc

N = 10000
E = 320000
D = 128
NG = 16
NCLS = 25

NC, NS = 2, 16
NW = NC * NS
EPT = E // NW
CHUNK = 125
NCHUNK = EPT // CHUNK
WBR = 80
NPAD = 10240
ZR = NPAD // NS

RB = 1000
GRID = N // RB


def _sc_degree(dst3, ones_h, zeros_h):
  mesh = plsc.VectorSubcoreMesh(core_axis_name="c", subcore_axis_name="s")

  @functools.partial(
      pl.kernel,
      out_type=jax.ShapeDtypeStruct((NC, NPAD, D), jnp.float32),
      mesh=mesh,
      scratch_types=[
          pltpu.VMEM_SHARED((NPAD, D), jnp.float32),
          pltpu.VMEM((NCHUNK, CHUNK), jnp.int32),
          pltpu.VMEM((CHUNK, D), jnp.float32),
          pltpu.SemaphoreType.DMA,
      ],
  )
  def k(dst_hbm, ones_hbm, zeros_hbm, out_hbm, acc_sh, dst_v, ones_v, sem):
    cid = lax.axis_index("c")
    sid = lax.axis_index("s")
    wid = sid * NC + cid
    pltpu.sync_copy(ones_hbm, ones_v)
    pltpu.sync_copy(zeros_hbm, acc_sh.at[pl.ds(sid * ZR, ZR)])
    plsc.subcore_barrier()
    pltpu.sync_copy(dst_hbm.at[wid], dst_v)

    @pl.loop(0, NCHUNK)
    def _(c):
      pltpu.async_copy(ones_v, acc_sh.at[dst_v.at[c]], sem, add=True)

    @pl.loop(0, NCHUNK)
    def _(c):
      pltpu.make_async_copy(ones_v, acc_sh.at[dst_v.at[c]], sem).wait()

    plsc.subcore_barrier()
    pltpu.sync_copy(acc_sh.at[pl.ds(sid * ZR, ZR)],
                    out_hbm.at[cid, pl.ds(sid * ZR, ZR)])

  return k(dst3, ones_h, zeros_h)


def _sc_message(hws, src3, dst4, zeros_h):
  mesh = plsc.VectorSubcoreMesh(core_axis_name="c", subcore_axis_name="s")

  @functools.partial(
      pl.kernel,
      out_type=jax.ShapeDtypeStruct((NC, NPAD, D), jnp.float32),
      mesh=mesh,
      scratch_types=[
          pltpu.VMEM_SHARED((NPAD, D), jnp.float32),
          pltpu.VMEM((NCHUNK, CHUNK), jnp.int32),
          pltpu.VMEM((1, CHUNK), jnp.int32),
          pltpu.VMEM((1, CHUNK), jnp.int32),
          pltpu.VMEM((CHUNK, D), jnp.float32),
          pltpu.VMEM((CHUNK, D), jnp.float32),
          pltpu.SemaphoreType.DMA,
          pltpu.SemaphoreType.DMA,
          pltpu.SemaphoreType.DMA,
          pltpu.SemaphoreType.DMA,
      ],
  )
  def k(hws_hbm, src_hbm, dst_hbm, out_hbm,
        acc_sh, src_v, dst_0, dst_1, msg_0, msg_1, g0, g1, s0, s1):
    cid = lax.axis_index("c")
    sid = lax.axis_index("s")
    wid = sid * NC + cid
    base = sid * ZR
    msgs = (msg_0, msg_1)
    dsts = (dst_0, dst_1)
    gsems = (g0, g1)
    ssems = (s0, s1)

    for r in range(16):
      for j in range(D // 16):
        msg_1[r, pl.ds(j * 16, 16)] = jnp.zeros((16,), jnp.float32)

    @pl.loop(0, ZR // 16)
    def _(t):
      pltpu.async_copy(msg_1.at[pl.ds(0, 16)],
                       acc_sh.at[pl.ds(base + t * 16, 16)], s1)

    pltpu.sync_copy(src_hbm.at[wid], src_v)

    def fetch(c, b):
      pltpu.async_copy(hws_hbm.at[src_v.at[c]], msgs[b], gsems[b])
      pltpu.async_copy(dst_hbm.at[wid, c], dsts[b], gsems[b])

    def wait_fetch(c, b):
      pltpu.make_async_copy(hws_hbm.at[src_v.at[c]], msgs[b], gsems[b]).wait()
      pltpu.make_async_copy(dst_hbm.at[wid, c], dsts[b], gsems[b]).wait()

    def scat(b):
      pltpu.async_copy(msgs[b], acc_sh.at[dsts[b].at[0]], ssems[b], add=True)

    def wait_scat(b):
      pltpu.make_async_copy(msgs[b], acc_sh.at[dsts[b].at[0]],
                            ssems[b]).wait()

    fetch(0, 0)

    @pl.loop(0, ZR // 16)
    def _(t):
      pltpu.make_async_copy(msg_1.at[pl.ds(0, 16)],
                            acc_sh.at[pl.ds(base, 16)], s1).wait()

    plsc.subcore_barrier()

    @pl.loop(0, NCHUNK - 1, step=2)
    def _(c):
      @pl.when(c >= 2)
      def _():
        wait_scat(1)

      fetch(c + 1, 1)
      wait_fetch(c, 0)
      scat(0)
      wait_scat(0)

      @pl.when(c < NCHUNK - 2)
      def _():
        fetch(c + 2, 0)

      wait_fetch(c + 1, 1)
      scat(1)

    wait_scat(1)

    plsc.subcore_barrier()

    def wb_in(t, b):
      pltpu.async_copy(acc_sh.at[pl.ds(base + t * WBR, WBR)],
                       msgs[b].at[pl.ds(0, WBR)], gsems[b])

    def wb_win(t, b):
      pltpu.make_async_copy(acc_sh.at[pl.ds(base + t * WBR, WBR)],
                            msgs[b].at[pl.ds(0, WBR)], gsems[b]).wait()

    def wb_out(t, b):
      pltpu.async_copy(msgs[b].at[pl.ds(0, WBR)],
                       out_hbm.at[cid, pl.ds(base + t * WBR, WBR)],
                       ssems[b])

    def wb_wout(t, b):
      pltpu.make_async_copy(msgs[b].at[pl.ds(0, WBR)],
                            out_hbm.at[cid, pl.ds(base + t * WBR, WBR)],
                            ssems[b]).wait()

    nwb = ZR // WBR
    wb_in(0, 0)
    wb_in(1, 1)
    for t in range(nwb):
      b = t % 2
      wb_win(t, b)
      wb_out(t, b)
      if t + 2 < nwb:
        wb_wout(t, b)
        wb_in(t + 2, b)
    for t in (nwb - 2, nwb - 1):
      wb_wout(t, t % 2)

  return k(hws, src3, dst4)


def _prep_body(x_ref, win_ref, bin_ref, w0_ref, degp_ref, hws_ref, dinv_ref):
  deg = degp_ref[0][:, 0:1] + degp_ref[1][:, 0:1] + 1.0
  dinv = lax.rsqrt(deg)
  h = jnp.maximum(jnp.dot(x_ref[...], win_ref[...],
                          preferred_element_type=jnp.float32) + bin_ref[...],
                  0.0)
  hws_ref[...] = jnp.dot(h, w0_ref[...],
                         preferred_element_type=jnp.float32) * dinv
  dinv_ref[...] = dinv


def _tc_prep(x, W_in, b_in2, W0, degp):
  row = lambda i: (i, 0)
  full = lambda i: (0, 0)
  return pl.pallas_call(
      _prep_body,
      grid=(GRID,),
      in_specs=[
          pl.BlockSpec((RB, D), row),
          pl.BlockSpec((D, D), full),
          pl.BlockSpec((1, D), full),
          pl.BlockSpec((D, D), full),
          pl.BlockSpec((NC, RB, D), lambda i: (0, i, 0)),
      ],
      out_specs=[pl.BlockSpec((RB, D), row), pl.BlockSpec((RB, 1), row)],
      out_shape=[
          jax.ShapeDtypeStruct((N, D), jnp.float32),
          jax.ShapeDtypeStruct((N, 1), jnp.float32),
      ],
  )(x, W_in, b_in2, W0, degp)


def _mid_body(p_ref, hws_ref, dinv_ref, b_ref, bns_ref, bnb_ref, wn_ref,
              out_ref):
  dinv = dinv_ref[...]
  o = (p_ref[0] + p_ref[1] + hws_ref[...]) * dinv + b_ref[...]
  h = jnp.maximum(o * bns_ref[...] + bnb_ref[...], 0.0)
  out_ref[...] = jnp.dot(h, wn_ref[...],
                         preferred_element_type=jnp.float32) * dinv


def _tc_mid(P, hws, dinv, b2, bns, bnb, Wn):
  row = lambda i: (i, 0)
  full = lambda i: (0, 0)
  return pl.pallas_call(
      _mid_body,
      grid=(GRID,),
      in_specs=[
          pl.BlockSpec((NC, RB, D), lambda i: (0, i, 0)),
          pl.BlockSpec((RB, D), row),
          pl.BlockSpec((RB, 1), row),
          pl.BlockSpec((1, D), full),
          pl.BlockSpec((1, D), full),
          pl.BlockSpec((1, D), full),
          pl.BlockSpec((D, D), full),
      ],
      out_specs=pl.BlockSpec((RB, D), row),
      out_shape=jax.ShapeDtypeStruct((N, D), jnp.float32),
  )(P, hws, dinv, b2, bns, bnb, Wn)


def _final_body(p_ref, hws_ref, dinv_ref, b_ref, bns_ref, bnb_ref,
                batch_ref,
                w1_ref, b1_ref, w2_ref, b2_ref, out_ref,
                sums, counts, maxes):
  i = pl.program_id(0)

  @pl.when(i == 0)
  def _():
    sums[...] = jnp.zeros_like(sums)
    counts[...] = jnp.zeros_like(counts)
    maxes[...] = jnp.full_like(maxes, -jnp.inf)

  o = (p_ref[0] + p_ref[1] + hws_ref[...]) * dinv_ref[...] + b_ref[...]
  h = jnp.maximum(o * bns_ref[...] + bnb_ref[...], 0.0)
  bcol = batch_ref[...]
  onehot = (bcol ==
            lax.broadcasted_iota(jnp.int32, (RB, NG), 1)).astype(jnp.float32)
  sums[...] += lax.dot_general(onehot, h, (((0,), (0,)), ((), ())),
                               preferred_element_type=jnp.float32)
  counts[...] += jnp.broadcast_to(jnp.sum(onehot, axis=0, keepdims=True).T,
                                  (NG, D))
  for g in range(NG):
    m = jnp.max(jnp.where(bcol == g, h, -jnp.inf), axis=0, keepdims=True)
    maxes[g:g + 1, :] = jnp.maximum(maxes[g:g + 1, :], m)

  @pl.when(i == GRID - 1)
  def _():
    mean = sums[...] / jnp.maximum(counts[...], 1.0)
    mx = maxes[...]
    mx = jnp.where(mx == -jnp.inf, 0.0, mx)
    z = jnp.concatenate([mean, mx], axis=1)
    hc = jnp.maximum(jnp.dot(z, w1_ref[...],
                             preferred_element_type=jnp.float32) + b1_ref[...],
                     0.0)
    out_ref[...] = jnp.dot(hc, w2_ref[...],
                           preferred_element_type=jnp.float32) + b2_ref[...]


def _tc_final(P, hws, dinv, b2, bns, bnb, batch2, W1, b1_2, W2, b2_2):
  row = lambda i: (i, 0)
  full = lambda i: (0, 0)
  return pl.pallas_call(
      _final_body,
      grid=(GRID,),
      in_specs=[
          pl.BlockSpec((NC, RB, D), lambda i: (0, i, 0)),
          pl.BlockSpec((RB, D), row),
          pl.BlockSpec((RB, 1), row),
          pl.BlockSpec((1, D), full),
          pl.BlockSpec((1, D), full),
          pl.BlockSpec((1, D), full),
          pl.BlockSpec((RB, 1), row),
          pl.BlockSpec((2 * D, D), full),
          pl.BlockSpec((1, D), full),
          pl.BlockSpec((D, NCLS), full),
          pl.BlockSpec((1, NCLS), full),
      ],
      out_specs=pl.BlockSpec((NG, NCLS), full),
      out_shape=jax.ShapeDtypeStruct((NG, NCLS), jnp.float32),
      scratch_shapes=[
          pltpu.VMEM((NG, D), jnp.float32),
          pltpu.VMEM((NG, D), jnp.float32),
          pltpu.VMEM((NG, D), jnp.float32),
      ],
  )(P, hws, dinv, b2, bns, bnb, batch2, W1, b1_2, W2, b2_2)


def kernel(x, edge_index, batch, W_in, b_in,
           conv_W0, conv_b0, conv_W1, conv_b1, conv_W2, conv_b2,
           bn_g0, bn_b0, bn_m0, bn_v0,
           bn_g1, bn_b1, bn_m1, bn_v1,
           bn_g2, bn_b2, bn_m2, bn_v2,
           cls_W1, cls_b1, cls_W2, cls_b2):
  src3 = edge_index[0].reshape(NW, NCHUNK, CHUNK)
  dst3 = edge_index[1].reshape(NW, NCHUNK, CHUNK)
  dst4 = edge_index[1].reshape(NW, NCHUNK, 1, CHUNK)
  batch2 = batch.reshape(N, 1)

  ones_h = jnp.ones((CHUNK, D), jnp.float32)
  zeros_h = jnp.zeros((ZR, D), jnp.float32)

  degp = _sc_degree(dst3, ones_h, zeros_h)

  bias = [conv_b0.reshape(1, D), conv_b1.reshape(1, D), conv_b2.reshape(1, D)]
  bns, bnb = [], []
  for g, bb, m, v in ((bn_g0, bn_b0, bn_m0, bn_v0),
                      (bn_g1, bn_b1, bn_m1, bn_v1),
                      (bn_g2, bn_b2, bn_m2, bn_v2)):
    s = g / jnp.sqrt(v + 1e-5)
    bns.append(s.reshape(1, D))
    bnb.append((bb - m * s).reshape(1, D))

  hws0, dinv = _tc_prep(x, W_in, b_in.reshape(1, D), conv_W0, degp)
  P0 = _sc_message(hws0, src3, dst4, zeros_h)
  hws1 = _tc_mid(P0, hws0, dinv, bias[0], bns[0], bnb[0], conv_W1)
  P1 = _sc_message(hws1, src3, dst4, zeros_h)
  hws2 = _tc_mid(P1, hws1, dinv, bias[1], bns[1], bnb[1], conv_W2)
  P2 = _sc_message(hws2, src3, dst4, zeros_h)
  out = _tc_final(P2, hws2, dinv, bias[2], bns[2], bnb[2], batch2,
                  cls_W1, cls_b1.reshape(1, D), cls_W2,
                  cls_b2.reshape(1, NCLS))
  return out


# speedup vs baseline: 1.0686x; 1.0686x over previous
import functools

import jax
import jax.numpy as jnp
from jax import lax
from jax.experimental import pallas as pl
from jax.experimental.pallas import tpu as pltpu
from jax.experimental.pallas import tpu_sc as plsc

N = 10000
E = 320000
D = 128
NG = 16
NCLS = 25

NC, NS = 2, 16
NW = NC * NS
EPT = E // NW
CHUNK = 80
NCHUNK = EPT // CHUNK
NPAD = 10240
ZR = NPAD // NS

RB = 1000
GRID = N // RB


def _sc_degree(dst3, ones_h, zeros_h):
  mesh = plsc.VectorSubcoreMesh(core_axis_name="c", subcore_axis_name="s")

  @functools.partial(
      pl.kernel,
      out_type=jax.ShapeDtypeStruct((NC, NPAD, D), jnp.float32),
      mesh=mesh,
      scratch_types=[
          pltpu.VMEM_SHARED((NPAD, D), jnp.float32),
          pltpu.VMEM((NCHUNK, CHUNK), jnp.int32),
          pltpu.VMEM((CHUNK, D), jnp.float32),
          pltpu.SemaphoreType.DMA,
      ],
  )
  def k(dst_hbm, ones_hbm, zeros_hbm, out_hbm, acc_sh, dst_v, ones_v, sem):
    cid = lax.axis_index("c")
    sid = lax.axis_index("s")
    wid = sid * NC + cid
    pltpu.sync_copy(ones_hbm, ones_v)
    pltpu.sync_copy(zeros_hbm, acc_sh.at[pl.ds(sid * ZR, ZR)])
    plsc.subcore_barrier()
    pltpu.sync_copy(dst_hbm.at[wid], dst_v)

    @pl.loop(0, NCHUNK)
    def _(c):
      pltpu.async_copy(ones_v, acc_sh.at[dst_v.at[c]], sem, add=True)

    @pl.loop(0, NCHUNK)
    def _(c):
      pltpu.make_async_copy(ones_v, acc_sh.at[dst_v.at[c]], sem).wait()

    plsc.subcore_barrier()
    pltpu.sync_copy(acc_sh.at[pl.ds(sid * ZR, ZR)],
                    out_hbm.at[cid, pl.ds(sid * ZR, ZR)])

  return k(dst3, ones_h, zeros_h)


def _sc_message(hws, src3, dst4, zeros_h):
  mesh = plsc.VectorSubcoreMesh(core_axis_name="c", subcore_axis_name="s")

  @functools.partial(
      pl.kernel,
      out_type=jax.ShapeDtypeStruct((NC, NPAD, D), jnp.float32),
      mesh=mesh,
      scratch_types=[
          pltpu.VMEM_SHARED((NPAD, D), jnp.float32),
          pltpu.VMEM((NCHUNK, CHUNK), jnp.int32),
          pltpu.VMEM((1, CHUNK), jnp.int32),
          pltpu.VMEM((1, CHUNK), jnp.int32),
          pltpu.VMEM((1, CHUNK), jnp.int32),
          pltpu.VMEM((CHUNK, D), jnp.float32),
          pltpu.VMEM((CHUNK, D), jnp.float32),
          pltpu.VMEM((CHUNK, D), jnp.float32),
          pltpu.SemaphoreType.DMA,
          pltpu.SemaphoreType.DMA,
          pltpu.SemaphoreType.DMA,
          pltpu.SemaphoreType.DMA,
          pltpu.SemaphoreType.DMA,
          pltpu.SemaphoreType.DMA,
      ],
  )
  def k(hws_hbm, src_hbm, dst_hbm, out_hbm,
        acc_sh, src_v, dst_0, dst_1, dst_2, msg_0, msg_1, msg_2,
        g0, g1, g2, s0, s1, s2):
    cid = lax.axis_index("c")
    sid = lax.axis_index("s")
    wid = sid * NC + cid
    base = sid * ZR
    msgs = (msg_0, msg_1, msg_2)
    dsts = (dst_0, dst_1, dst_2)
    gsems = (g0, g1, g2)
    ssems = (s0, s1, s2)

    for r in range(16):
      for j in range(D // 16):
        msg_2[r, pl.ds(j * 16, 16)] = jnp.zeros((16,), jnp.float32)

    @pl.loop(0, ZR // 16)
    def _(t):
      pltpu.async_copy(msg_2.at[pl.ds(0, 16)],
                       acc_sh.at[pl.ds(base + t * 16, 16)], s2)

    pltpu.sync_copy(src_hbm.at[wid], src_v)

    def fetch(c, b):
      pltpu.async_copy(hws_hbm.at[src_v.at[c]], msgs[b], gsems[b])
      pltpu.async_copy(dst_hbm.at[wid, c], dsts[b], gsems[b])

    def wait_fetch(c, b):
      pltpu.make_async_copy(hws_hbm.at[src_v.at[c]], msgs[b], gsems[b]).wait()
      pltpu.make_async_copy(dst_hbm.at[wid, c], dsts[b], gsems[b]).wait()

    def scat(b):
      pltpu.async_copy(msgs[b], acc_sh.at[dsts[b].at[0]], ssems[b], add=True)

    def wait_scat(b):
      pltpu.make_async_copy(msgs[b], acc_sh.at[dsts[b].at[0]],
                            ssems[b]).wait()

    fetch(0, 0)
    fetch(1, 1)

    @pl.loop(0, ZR // 16)
    def _(t):
      pltpu.make_async_copy(msg_2.at[pl.ds(0, 16)],
                            acc_sh.at[pl.ds(base, 16)], s2).wait()

    plsc.subcore_barrier()

    @pl.loop(0, NCHUNK - 2, step=3)
    def _(c):
      for k3 in range(3):
        bpf = (k3 + 2) % 3

        @pl.when(c + k3 >= 1)
        def _():
          wait_scat(bpf)

        fetch(c + k3 + 2, bpf)
        wait_fetch(c + k3, k3)
        scat(k3)

    for ch in (NCHUNK - 2, NCHUNK - 1):
      b = ch % 3
      wait_fetch(ch, b)
      scat(b)
    for b in ((NCHUNK - 3) % 3, (NCHUNK - 2) % 3, (NCHUNK - 1) % 3):
      wait_scat(b)

    plsc.subcore_barrier()

    def wb_in(t, b):
      pltpu.async_copy(acc_sh.at[pl.ds(base + t * CHUNK, CHUNK)],
                       msgs[b], gsems[b])

    def wb_win(t, b):
      pltpu.make_async_copy(acc_sh.at[pl.ds(base + t * CHUNK, CHUNK)],
                            msgs[b], gsems[b]).wait()

    def wb_out(t, b):
      pltpu.async_copy(msgs[b],
                       out_hbm.at[cid, pl.ds(base + t * CHUNK, CHUNK)],
                       ssems[b])

    def wb_wout(t, b):
      pltpu.make_async_copy(msgs[b],
                            out_hbm.at[cid, pl.ds(base + t * CHUNK, CHUNK)],
                            ssems[b]).wait()

    nwb = ZR // CHUNK
    wb_in(0, 0)
    wb_in(1, 1)
    wb_in(2, 2)
    for t in range(nwb):
      b = t % 3
      wb_win(t, b)
      wb_out(t, b)
      if t + 3 < nwb:
        wb_wout(t, b)
        wb_in(t + 3, b)
    for t in (nwb - 3, nwb - 2, nwb - 1):
      wb_wout(t, t % 3)

  return k(hws, src3, dst4)


def _prep_body(x_ref, win_ref, bin_ref, w0_ref, degp_ref, hws_ref, dinv_ref):
  deg = degp_ref[0][:, 0:1] + degp_ref[1][:, 0:1] + 1.0
  dinv = lax.rsqrt(deg)
  h = jnp.maximum(jnp.dot(x_ref[...], win_ref[...],
                          preferred_element_type=jnp.float32) + bin_ref[...],
                  0.0)
  hws_ref[...] = jnp.dot(h, w0_ref[...],
                         preferred_element_type=jnp.float32) * dinv
  dinv_ref[...] = dinv


def _tc_prep(x, W_in, b_in2, W0, degp):
  row = lambda i: (i, 0)
  full = lambda i: (0, 0)
  return pl.pallas_call(
      _prep_body,
      grid=(GRID,),
      in_specs=[
          pl.BlockSpec((RB, D), row),
          pl.BlockSpec((D, D), full),
          pl.BlockSpec((1, D), full),
          pl.BlockSpec((D, D), full),
          pl.BlockSpec((NC, RB, D), lambda i: (0, i, 0)),
      ],
      out_specs=[pl.BlockSpec((RB, D), row), pl.BlockSpec((RB, 1), row)],
      out_shape=[
          jax.ShapeDtypeStruct((N, D), jnp.float32),
          jax.ShapeDtypeStruct((N, 1), jnp.float32),
      ],
  )(x, W_in, b_in2, W0, degp)


def _mid_body(p_ref, hws_ref, dinv_ref, b_ref, bns_ref, bnb_ref, wn_ref,
              out_ref):
  dinv = dinv_ref[...]
  o = (p_ref[0] + p_ref[1] + hws_ref[...]) * dinv + b_ref[...]
  h = jnp.maximum(o * bns_ref[...] + bnb_ref[...], 0.0)
  out_ref[...] = jnp.dot(h, wn_ref[...],
                         preferred_element_type=jnp.float32) * dinv


def _tc_mid(P, hws, dinv, b2, bns, bnb, Wn):
  row = lambda i: (i, 0)
  full = lambda i: (0, 0)
  return pl.pallas_call(
      _mid_body,
      grid=(GRID,),
      in_specs=[
          pl.BlockSpec((NC, RB, D), lambda i: (0, i, 0)),
          pl.BlockSpec((RB, D), row),
          pl.BlockSpec((RB, 1), row),
          pl.BlockSpec((1, D), full),
          pl.BlockSpec((1, D), full),
          pl.BlockSpec((1, D), full),
          pl.BlockSpec((D, D), full),
      ],
      out_specs=pl.BlockSpec((RB, D), row),
      out_shape=jax.ShapeDtypeStruct((N, D), jnp.float32),
  )(P, hws, dinv, b2, bns, bnb, Wn)


def _final_body(p_ref, hws_ref, dinv_ref, b_ref, bns_ref, bnb_ref,
                batch_ref,
                w1_ref, b1_ref, w2_ref, b2_ref, out_ref,
                sums, counts, maxes):
  i = pl.program_id(0)

  @pl.when(i == 0)
  def _():
    sums[...] = jnp.zeros_like(sums)
    counts[...] = jnp.zeros_like(counts)
    maxes[...] = jnp.full_like(maxes, -jnp.inf)

  o = (p_ref[0] + p_ref[1] + hws_ref[...]) * dinv_ref[...] + b_ref[...]
  h = jnp.maximum(o * bns_ref[...] + bnb_ref[...], 0.0)
  bcol = batch_ref[...]
  onehot = (bcol ==
            lax.broadcasted_iota(jnp.int32, (RB, NG), 1)).astype(jnp.float32)
  sums[...] += lax.dot_general(onehot, h, (((0,), (0,)), ((), ())),
                               preferred_element_type=jnp.float32)
  counts[...] += jnp.broadcast_to(jnp.sum(onehot, axis=0, keepdims=True).T,
                                  (NG, D))
  for g in range(NG):
    m = jnp.max(jnp.where(bcol == g, h, -jnp.inf), axis=0, keepdims=True)
    maxes[g:g + 1, :] = jnp.maximum(maxes[g:g + 1, :], m)

  @pl.when(i == GRID - 1)
  def _():
    mean = sums[...] / jnp.maximum(counts[...], 1.0)
    mx = maxes[...]
    mx = jnp.where(mx == -jnp.inf, 0.0, mx)
    z = jnp.concatenate([mean, mx], axis=1)
    hc = jnp.maximum(jnp.dot(z, w1_ref[...],
                             preferred_element_type=jnp.float32) + b1_ref[...],
                     0.0)
    out_ref[...] = jnp.dot(hc, w2_ref[...],
                           preferred_element_type=jnp.float32) + b2_ref[...]


def _tc_final(P, hws, dinv, b2, bns, bnb, batch2, W1, b1_2, W2, b2_2):
  row = lambda i: (i, 0)
  full = lambda i: (0, 0)
  return pl.pallas_call(
      _final_body,
      grid=(GRID,),
      in_specs=[
          pl.BlockSpec((NC, RB, D), lambda i: (0, i, 0)),
          pl.BlockSpec((RB, D), row),
          pl.BlockSpec((RB, 1), row),
          pl.BlockSpec((1, D), full),
          pl.BlockSpec((1, D), full),
          pl.BlockSpec((1, D), full),
          pl.BlockSpec((RB, 1), row),
          pl.BlockSpec((2 * D, D), full),
          pl.BlockSpec((1, D), full),
          pl.BlockSpec((D, NCLS), full),
          pl.BlockSpec((1, NCLS), full),
      ],
      out_specs=pl.BlockSpec((NG, NCLS), full),
      out_shape=jax.ShapeDtypeStruct((NG, NCLS), jnp.float32),
      scratch_shapes=[
          pltpu.VMEM((NG, D), jnp.float32),
          pltpu.VMEM((NG, D), jnp.float32),
          pltpu.VMEM((NG, D), jnp.float32),
      ],
  )(P, hws, dinv, b2, bns, bnb, batch2, W1, b1_2, W2, b2_2)


def kernel(x, edge_index, batch, W_in, b_in,
           conv_W0, conv_b0, conv_W1, conv_b1, conv_W2, conv_b2,
           bn_g0, bn_b0, bn_m0, bn_v0,
           bn_g1, bn_b1, bn_m1, bn_v1,
           bn_g2, bn_b2, bn_m2, bn_v2,
           cls_W1, cls_b1, cls_W2, cls_b2):
  src3 = edge_index[0].reshape(NW, NCHUNK, CHUNK)
  dst3 = edge_index[1].reshape(NW, NCHUNK, CHUNK)
  dst4 = edge_index[1].reshape(NW, NCHUNK, 1, CHUNK)
  batch2 = batch.reshape(N, 1)

  ones_h = jnp.ones((CHUNK, D), jnp.float32)
  zeros_h = jnp.zeros((ZR, D), jnp.float32)

  degp = _sc_degree(dst3, ones_h, zeros_h)

  bias = [conv_b0.reshape(1, D), conv_b1.reshape(1, D), conv_b2.reshape(1, D)]
  bns, bnb = [], []
  for g, bb, m, v in ((bn_g0, bn_b0, bn_m0, bn_v0),
                      (bn_g1, bn_b1, bn_m1, bn_v1),
                      (bn_g2, bn_b2, bn_m2, bn_v2)):
    s = g / jnp.sqrt(v + 1e-5)
    bns.append(s.reshape(1, D))
    bnb.append((bb - m * s).reshape(1, D))

  hws0, dinv = _tc_prep(x, W_in, b_in.reshape(1, D), conv_W0, degp)
  P0 = _sc_message(hws0, src3, dst4, zeros_h)
  hws1 = _tc_mid(P0, hws0, dinv, bias[0], bns[0], bnb[0], conv_W1)
  P1 = _sc_message(hws1, src3, dst4, zeros_h)
  hws2 = _tc_mid(P1, hws1, dinv, bias[1], bns[1], bnb[1], conv_W2)
  P2 = _sc_message(hws2, src3, dst4, zeros_h)
  out = _tc_final(P2, hws2, dinv, bias[2], bns[2], bnb[2], batch2,
                  cls_W1, cls_b1.reshape(1, D), cls_W2,
                  cls_b2.reshape(1, NCLS))
  return out


# speedup vs baseline: 1.0742x; 1.0052x over previous
import functools

import jax
import jax.numpy as jnp
from jax import lax
from jax.experimental import pallas as pl
from jax.experimental.pallas import tpu as pltpu
from jax.experimental.pallas import tpu_sc as plsc

N = 10000
E = 320000
D = 128
NG = 16
NCLS = 25

NC, NS = 2, 16
NW = NC * NS
EPT = E // NW
CHUNK = 80
NCHUNK = EPT // CHUNK
NPAD = 10240
ZR = NPAD // NS

RB = 1000
GRID = N // RB


def _sc_degree(dst3, ones_h, zeros_h):
  mesh = plsc.VectorSubcoreMesh(core_axis_name="c", subcore_axis_name="s")

  @functools.partial(
      pl.kernel,
      out_type=jax.ShapeDtypeStruct((NC, NPAD, D), jnp.float32),
      mesh=mesh,
      scratch_types=[
          pltpu.VMEM_SHARED((NPAD, D), jnp.float32),
          pltpu.VMEM((NCHUNK, CHUNK), jnp.int32),
          pltpu.VMEM((CHUNK, D), jnp.float32),
          pltpu.SemaphoreType.DMA,
          pltpu.SemaphoreType.DMA,
      ],
  )
  def k(dst_hbm, ones_hbm, zeros_hbm, out_hbm, acc_sh, dst_v, ones_v, sem,
        zsem):
    cid = lax.axis_index("c")
    sid = lax.axis_index("s")
    wid = sid * NC + cid
    pltpu.async_copy(zeros_hbm, acc_sh.at[pl.ds(sid * ZR, ZR)], zsem)
    pltpu.sync_copy(ones_hbm, ones_v)
    pltpu.sync_copy(dst_hbm.at[wid], dst_v)
    pltpu.make_async_copy(zeros_hbm, acc_sh.at[pl.ds(sid * ZR, ZR)],
                          zsem).wait()
    plsc.subcore_barrier()

    @pl.loop(0, NCHUNK)
    def _(c):
      pltpu.async_copy(ones_v, acc_sh.at[dst_v.at[c]], sem, add=True)

    @pl.loop(0, NCHUNK)
    def _(c):
      pltpu.make_async_copy(ones_v, acc_sh.at[dst_v.at[c]], sem).wait()

    plsc.subcore_barrier()
    pltpu.sync_copy(acc_sh.at[pl.ds(sid * ZR, ZR)],
                    out_hbm.at[cid, pl.ds(sid * ZR, ZR)])

  return k(dst3, ones_h, zeros_h)


def _sc_message(hws, src3, dst4, zeros_h):
  mesh = plsc.VectorSubcoreMesh(core_axis_name="c", subcore_axis_name="s")

  @functools.partial(
      pl.kernel,
      out_type=jax.ShapeDtypeStruct((NC, NPAD, D), jnp.float32),
      mesh=mesh,
      scratch_types=[
          pltpu.VMEM_SHARED((NPAD, D), jnp.float32),
          pltpu.VMEM((NCHUNK, CHUNK), jnp.int32),
          pltpu.VMEM((1, CHUNK), jnp.int32),
          pltpu.VMEM((1, CHUNK), jnp.int32),
          pltpu.VMEM((1, CHUNK), jnp.int32),
          pltpu.VMEM((CHUNK, D), jnp.float32),
          pltpu.VMEM((CHUNK, D), jnp.float32),
          pltpu.VMEM((CHUNK, D), jnp.float32),
          pltpu.SemaphoreType.DMA,
          pltpu.SemaphoreType.DMA,
          pltpu.SemaphoreType.DMA,
          pltpu.SemaphoreType.DMA,
          pltpu.SemaphoreType.DMA,
          pltpu.SemaphoreType.DMA,
      ],
  )
  def k(hws_hbm, src_hbm, dst_hbm, out_hbm,
        acc_sh, src_v, dst_0, dst_1, dst_2, msg_0, msg_1, msg_2,
        g0, g1, g2, s0, s1, s2):
    cid = lax.axis_index("c")
    sid = lax.axis_index("s")
    wid = sid * NC + cid
    base = sid * ZR
    msgs = (msg_0, msg_1, msg_2)
    dsts = (dst_0, dst_1, dst_2)
    gsems = (g0, g1, g2)
    ssems = (s0, s1, s2)

    for r in range(16):
      for j in range(D // 16):
        msg_2[r, pl.ds(j * 16, 16)] = jnp.zeros((16,), jnp.float32)

    @pl.loop(0, ZR // 16)
    def _(t):
      pltpu.async_copy(msg_2.at[pl.ds(0, 16)],
                       acc_sh.at[pl.ds(base + t * 16, 16)], s2)

    pltpu.sync_copy(src_hbm.at[wid], src_v)

    def fetch(c, b):
      pltpu.async_copy(hws_hbm.at[src_v.at[c]], msgs[b], gsems[b])
      pltpu.async_copy(dst_hbm.at[wid, c], dsts[b], gsems[b])

    def wait_fetch(c, b):
      pltpu.make_async_copy(hws_hbm.at[src_v.at[c]], msgs[b], gsems[b]).wait()
      pltpu.make_async_copy(dst_hbm.at[wid, c], dsts[b], gsems[b]).wait()

    def scat(b):
      pltpu.async_copy(msgs[b], acc_sh.at[dsts[b].at[0]], ssems[b], add=True)

    def wait_scat(b):
      pltpu.make_async_copy(msgs[b], acc_sh.at[dsts[b].at[0]],
                            ssems[b]).wait()

    fetch(0, 0)
    fetch(1, 1)

    @pl.loop(0, ZR // 16)
    def _(t):
      pltpu.make_async_copy(msg_2.at[pl.ds(0, 16)],
                            acc_sh.at[pl.ds(base, 16)], s2).wait()

    plsc.subcore_barrier()

    @pl.loop(0, NCHUNK - 2, step=3)
    def _(c):
      for k3 in range(3):
        bpf = (k3 + 2) % 3

        @pl.when(c + k3 >= 1)
        def _():
          wait_scat(bpf)

        fetch(c + k3 + 2, bpf)
        wait_fetch(c + k3, k3)
        scat(k3)

    for ch in (NCHUNK - 2, NCHUNK - 1):
      b = ch % 3
      wait_fetch(ch, b)
      scat(b)
    for b in ((NCHUNK - 3) % 3, (NCHUNK - 2) % 3, (NCHUNK - 1) % 3):
      wait_scat(b)

    plsc.subcore_barrier()

    def wb_in(t, b):
      pltpu.async_copy(acc_sh.at[pl.ds(base + t * CHUNK, CHUNK)],
                       msgs[b], gsems[b])

    def wb_win(t, b):
      pltpu.make_async_copy(acc_sh.at[pl.ds(base + t * CHUNK, CHUNK)],
                            msgs[b], gsems[b]).wait()

    def wb_out(t, b):
      pltpu.async_copy(msgs[b],
                       out_hbm.at[cid, pl.ds(base + t * CHUNK, CHUNK)],
                       ssems[b])

    def wb_wout(t, b):
      pltpu.make_async_copy(msgs[b],
                            out_hbm.at[cid, pl.ds(base + t * CHUNK, CHUNK)],
                            ssems[b]).wait()

    nwb = ZR // CHUNK
    wb_in(0, 0)
    wb_in(1, 1)
    wb_in(2, 2)
    for t in range(nwb):
      b = t % 3
      wb_win(t, b)
      wb_out(t, b)
      if t + 3 < nwb:
        wb_wout(t, b)
        wb_in(t + 3, b)
    for t in (nwb - 3, nwb - 2, nwb - 1):
      wb_wout(t, t % 3)

  return k(hws, src3, dst4)


def _prep_body(x_ref, win_ref, bin_ref, w0_ref, degp_ref, hws_ref, dinv_ref):
  deg = degp_ref[0][:, 0:1] + degp_ref[1][:, 0:1] + 1.0
  dinv = lax.rsqrt(deg)
  h = jnp.maximum(jnp.dot(x_ref[...], win_ref[...],
                          preferred_element_type=jnp.float32) + bin_ref[...],
                  0.0)
  hws_ref[...] = jnp.dot(h, w0_ref[...],
                         preferred_element_type=jnp.float32) * dinv
  dinv_ref[...] = dinv


def _tc_prep(x, W_in, b_in2, W0, degp):
  row = lambda i: (i, 0)
  full = lambda i: (0, 0)
  return pl.pallas_call(
      _prep_body,
      grid=(GRID,),
      in_specs=[
          pl.BlockSpec((RB, D), row),
          pl.BlockSpec((D, D), full),
          pl.BlockSpec((1, D), full),
          pl.BlockSpec((D, D), full),
          pl.BlockSpec((NC, RB, D), lambda i: (0, i, 0)),
      ],
      out_specs=[pl.BlockSpec((RB, D), row), pl.BlockSpec((RB, 1), row)],
      out_shape=[
          jax.ShapeDtypeStruct((N, D), jnp.float32),
          jax.ShapeDtypeStruct((N, 1), jnp.float32),
      ],
  )(x, W_in, b_in2, W0, degp)


def _mid_body(p_ref, hws_ref, dinv_ref, b_ref, bns_ref, bnb_ref, wn_ref,
              out_ref):
  dinv = dinv_ref[...]
  o = (p_ref[0] + p_ref[1] + hws_ref[...]) * dinv + b_ref[...]
  h = jnp.maximum(o * bns_ref[...] + bnb_ref[...], 0.0)
  out_ref[...] = jnp.dot(h, wn_ref[...],
                         preferred_element_type=jnp.float32) * dinv


def _tc_mid(P, hws, dinv, b2, bns, bnb, Wn):
  row = lambda i: (i, 0)
  full = lambda i: (0, 0)
  return pl.pallas_call(
      _mid_body,
      grid=(GRID,),
      in_specs=[
          pl.BlockSpec((NC, RB, D), lambda i: (0, i, 0)),
          pl.BlockSpec((RB, D), row),
          pl.BlockSpec((RB, 1), row),
          pl.BlockSpec((1, D), full),
          pl.BlockSpec((1, D), full),
          pl.BlockSpec((1, D), full),
          pl.BlockSpec((D, D), full),
      ],
      out_specs=pl.BlockSpec((RB, D), row),
      out_shape=jax.ShapeDtypeStruct((N, D), jnp.float32),
  )(P, hws, dinv, b2, bns, bnb, Wn)


def _final_body(p_ref, hws_ref, dinv_ref, b_ref, bns_ref, bnb_ref,
                batch_ref,
                w1_ref, b1_ref, w2_ref, b2_ref, out_ref,
                sums, counts, maxes):
  i = pl.program_id(0)

  @pl.when(i == 0)
  def _():
    sums[...] = jnp.zeros_like(sums)
    counts[...] = jnp.zeros_like(counts)
    maxes[...] = jnp.full_like(maxes, -jnp.inf)

  o = (p_ref[0] + p_ref[1] + hws_ref[...]) * dinv_ref[...] + b_ref[...]
  h = jnp.maximum(o * bns_ref[...] + bnb_ref[...], 0.0)
  bcol = batch_ref[...]
  onehot = (bcol ==
            lax.broadcasted_iota(jnp.int32, (RB, NG), 1)).astype(jnp.float32)
  sums[...] += lax.dot_general(onehot, h, (((0,), (0,)), ((), ())),
                               preferred_element_type=jnp.float32)
  counts[...] += jnp.broadcast_to(jnp.sum(onehot, axis=0, keepdims=True).T,
                                  (NG, D))
  for g in range(NG):
    m = jnp.max(jnp.where(bcol == g, h, -jnp.inf), axis=0, keepdims=True)
    maxes[g:g + 1, :] = jnp.maximum(maxes[g:g + 1, :], m)

  @pl.when(i == GRID - 1)
  def _():
    mean = sums[...] / jnp.maximum(counts[...], 1.0)
    mx = maxes[...]
    mx = jnp.where(mx == -jnp.inf, 0.0, mx)
    z = jnp.concatenate([mean, mx], axis=1)
    hc = jnp.maximum(jnp.dot(z, w1_ref[...],
                             preferred_element_type=jnp.float32) + b1_ref[...],
                     0.0)
    out_ref[...] = jnp.dot(hc, w2_ref[...],
                           preferred_element_type=jnp.float32) + b2_ref[...]


def _tc_final(P, hws, dinv, b2, bns, bnb, batch2, W1, b1_2, W2, b2_2):
  row = lambda i: (i, 0)
  full = lambda i: (0, 0)
  return pl.pallas_call(
      _final_body,
      grid=(GRID,),
      in_specs=[
          pl.BlockSpec((NC, RB, D), lambda i: (0, i, 0)),
          pl.BlockSpec((RB, D), row),
          pl.BlockSpec((RB, 1), row),
          pl.BlockSpec((1, D), full),
          pl.BlockSpec((1, D), full),
          pl.BlockSpec((1, D), full),
          pl.BlockSpec((RB, 1), row),
          pl.BlockSpec((2 * D, D), full),
          pl.BlockSpec((1, D), full),
          pl.BlockSpec((D, NCLS), full),
          pl.BlockSpec((1, NCLS), full),
      ],
      out_specs=pl.BlockSpec((NG, NCLS), full),
      out_shape=jax.ShapeDtypeStruct((NG, NCLS), jnp.float32),
      scratch_shapes=[
          pltpu.VMEM((NG, D), jnp.float32),
          pltpu.VMEM((NG, D), jnp.float32),
          pltpu.VMEM((NG, D), jnp.float32),
      ],
  )(P, hws, dinv, b2, bns, bnb, batch2, W1, b1_2, W2, b2_2)


def kernel(x, edge_index, batch, W_in, b_in,
           conv_W0, conv_b0, conv_W1, conv_b1, conv_W2, conv_b2,
           bn_g0, bn_b0, bn_m0, bn_v0,
           bn_g1, bn_b1, bn_m1, bn_v1,
           bn_g2, bn_b2, bn_m2, bn_v2,
           cls_W1, cls_b1, cls_W2, cls_b2):
  src3 = edge_index[0].reshape(NW, NCHUNK, CHUNK)
  dst3 = edge_index[1].reshape(NW, NCHUNK, CHUNK)
  dst4 = edge_index[1].reshape(NW, NCHUNK, 1, CHUNK)
  batch2 = batch.reshape(N, 1)

  ones_h = jnp.ones((CHUNK, D), jnp.float32)
  zeros_h = jnp.zeros((ZR, D), jnp.float32)

  degp = _sc_degree(dst3, ones_h, zeros_h)

  bias = [conv_b0.reshape(1, D), conv_b1.reshape(1, D), conv_b2.reshape(1, D)]
  bns, bnb = [], []
  for g, bb, m, v in ((bn_g0, bn_b0, bn_m0, bn_v0),
                      (bn_g1, bn_b1, bn_m1, bn_v1),
                      (bn_g2, bn_b2, bn_m2, bn_v2)):
    s = g / jnp.sqrt(v + 1e-5)
    bns.append(s.reshape(1, D))
    bnb.append((bb - m * s).reshape(1, D))

  hws0, dinv = _tc_prep(x, W_in, b_in.reshape(1, D), conv_W0, degp)
  P0 = _sc_message(hws0, src3, dst4, zeros_h)
  hws1 = _tc_mid(P0, hws0, dinv, bias[0], bns[0], bnb[0], conv_W1)
  P1 = _sc_message(hws1, src3, dst4, zeros_h)
  hws2 = _tc_mid(P1, hws1, dinv, bias[1], bns[1], bnb[1], conv_W2)
  P2 = _sc_message(hws2, src3, dst4, zeros_h)
  out = _tc_final(P2, hws2, dinv, bias[2], bns[2], bnb[2], batch2,
                  cls_W1, cls_b1.reshape(1, D), cls_W2,
                  cls_b2.reshape(1, NCLS))
  return out
